# baseline (speedup 1.0000x reference)
import functools

import jax
import jax.numpy as jnp
from jax.experimental import pallas as pl

EB = 2


def _ffn_body(x_ref, tw_ref, ti_ref, w1_ref, w2_ref, out_ref):
    g_id = pl.program_id(0)
    F = w2_ref.shape[2]

    @pl.when(g_id == 0)
    def _():
        out_ref[...] = jnp.zeros_like(out_ref)

    x = x_ref[...]
    ids = ti_ref[...]
    tw = tw_ref[...]

    for j in range(EB):
        e = g_id * EB + j
        wg = w1_ref[j, :F]
        wu = w1_ref[j, F:]
        g = jax.lax.dot_general(x, wg, (((1,), (1,)), ((), ())),
                                preferred_element_type=jnp.float32)
        u = jax.lax.dot_general(x, wu, (((1,), (1,)), ((), ())),
                                preferred_element_type=jnp.float32)
        act = (g * jax.nn.sigmoid(g)) * u
        w2c = w2_ref[j]
        y = jax.lax.dot_general(act, w2c, (((1,), (1,)), ((), ())),
                                preferred_element_type=jnp.float32)
        wvec = jnp.sum(jnp.where(ids == e, tw, 0.0), axis=1)
        out_ref[...] += wvec[:, None] * y


@jax.jit
def kernel(hidden_states, topk_weights, topk_ids, w1, w2):
    T, D = hidden_states.shape
    E = w1.shape[0]
    F = w1.shape[1] // 2

    grid = (E // EB,)
    out = pl.pallas_call(
        _ffn_body,
        grid=grid,
        in_specs=[
            pl.BlockSpec((T, D), lambda i: (0, 0)),
            pl.BlockSpec(topk_weights.shape, lambda i: (0, 0)),
            pl.BlockSpec(topk_ids.shape, lambda i: (0, 0)),
            pl.BlockSpec((EB, 2 * F, D), lambda i: (i, 0, 0)),
            pl.BlockSpec((EB, D, F), lambda i: (i, 0, 0)),
        ],
        out_specs=pl.BlockSpec((T, D), lambda i: (0, 0)),
        out_shape=jax.ShapeDtypeStruct((T, D), jnp.float32),
    )(hidden_states, topk_weights, topk_ids, w1, w2)
    return out


# speedup vs baseline: 1.0710x; 1.0710x over previous
import functools

import jax
import jax.numpy as jnp
from jax.experimental import pallas as pl


def _ffn_body(x_ref, tw_ref, ti_ref, wg_ref, wu_ref, w2a_ref, w2b_ref, out_ref):
    e = pl.program_id(0)

    @pl.when(e == 0)
    def _():
        out_ref[...] = jnp.zeros_like(out_ref)

    x = x_ref[...]
    wg = wg_ref[0, 0]
    wu = wu_ref[0, 0]
    g = jax.lax.dot_general(x, wg, (((1,), (1,)), ((), ())),
                            preferred_element_type=jnp.float32)
    u = jax.lax.dot_general(x, wu, (((1,), (1,)), ((), ())),
                            preferred_element_type=jnp.float32)
    act = (g * jax.nn.sigmoid(g)) * u
    ya = jax.lax.dot_general(act, w2a_ref[0], (((1,), (1,)), ((), ())),
                             preferred_element_type=jnp.float32)
    yb = jax.lax.dot_general(act, w2b_ref[0], (((1,), (1,)), ((), ())),
                             preferred_element_type=jnp.float32)

    ids = ti_ref[...]
    tw = tw_ref[...]
    wvec = jnp.sum(jnp.where(ids == e, tw, 0.0), axis=1)[:, None]
    Dh = ya.shape[1]
    out_ref[:, :Dh] += wvec * ya
    out_ref[:, Dh:] += wvec * yb


@jax.jit
def kernel(hidden_states, topk_weights, topk_ids, w1, w2):
    T, D = hidden_states.shape
    E = w1.shape[0]
    F = w1.shape[1] // 2

    w1r = w1.reshape(E, 2, F, D)

    grid = (E,)
    out = pl.pallas_call(
        _ffn_body,
        grid=grid,
        in_specs=[
            pl.BlockSpec((T, D), lambda e: (0, 0)),
            pl.BlockSpec(topk_weights.shape, lambda e: (0, 0)),
            pl.BlockSpec(topk_ids.shape, lambda e: (0, 0)),
            pl.BlockSpec((1, 1, F, D), lambda e: (e, 0, 0, 0)),
            pl.BlockSpec((1, 1, F, D), lambda e: (e, 1, 0, 0)),
            pl.BlockSpec((1, D // 2, F), lambda e: (e, 0, 0)),
            pl.BlockSpec((1, D // 2, F), lambda e: (e, 1, 0)),
        ],
        out_specs=pl.BlockSpec((T, D), lambda e: (0, 0)),
        out_shape=jax.ShapeDtypeStruct((T, D), jnp.float32),
    )(hidden_states, topk_weights, topk_ids, w1r, w1r, w2, w2)
    return out
